# baseline (speedup 1.0000x reference)
import dataclasses
import functools

import jax
import jax.numpy as jnp
from jax import lax
from jax.experimental import pallas as pl
from jax.experimental.pallas import tpu as pltpu
from jax.experimental.pallas import tpu_sc as plsc

N = 10000
E = 320000
D_IN = 128
D_HID = 256
D_OUT = 128

NC, NS = 2, 16
BLK = 128
EB = 2560
E_PAD = EB * BLK
N_TAB = 10240
N_BINS = 10240

_mesh = plsc.VectorSubcoreMesh(
    core_axis_name="c", subcore_axis_name="s", num_cores=NC, num_subcores=NS)

_sc_params = pltpu.CompilerParams()
if "needs_layout_passes" in pltpu.CompilerParams.__dataclass_fields__:
  _sc_params = dataclasses.replace(_sc_params, needs_layout_passes=False)


def _hist_body(idx_hbm, out_hbm, idxbuf, histbuf):
  c = lax.axis_index("c")
  s = lax.axis_index("s")
  rpt = EB // NS
  ones16 = jnp.full((16,), 1.0, jnp.float32)

  for k in (0, 1):
    @pl.loop(0, N_BINS // 16)
    def _(i):
      histbuf[pl.ds(i * 16, 16)] = jnp.zeros((16,), jnp.float32)

    pltpu.sync_copy(idx_hbm.at[c, k, pl.ds(s * rpt, rpt)], idxbuf)

    @pl.loop(0, rpt * 8)
    def _(i):
      idx16 = idxbuf[i // 8, pl.ds((i % 8) * 16, 16)]
      plsc.addupdate_scatter(histbuf, [idx16], ones16)

    pltpu.sync_copy(histbuf, out_hbm.at[c, k, s])


def _sc_hist(idx_all):
  kern = pl.kernel(
      _hist_body,
      out_type=jax.ShapeDtypeStruct((2, 2, NS, N_BINS), jnp.float32),
      mesh=_mesh,
      scratch_types=[
          pltpu.VMEM((EB // NS, BLK), jnp.int32),
          pltpu.VMEM((N_BINS,), jnp.float32),
      ],
      compiler_params=_sc_params,
  )
  return kern(idx_all)


NHALF = N_TAB // 2
ACC_ROWS = 5376


def _make_scatter_body(n_tabs):
  rpt = EB // NS
  ch = rpt // 2

  def body(tab_hbm, src_hbm, dst_hbm, out_hbm, accum, gsem):
    pl.run_scoped(
        functools.partial(_scatter_inner, tab_hbm, src_hbm, dst_hbm, out_hbm,
                          accum, gsem),
        pltpu.VMEM((ch, BLK), jnp.int32),
        pltpu.VMEM((ch, BLK), jnp.int32),
        pltpu.VMEM((4, BLK, 128), jnp.float32),
        pltpu.VMEM((16, 128), jnp.float32),
    )

  def _scatter_inner(tab_hbm, src_hbm, dst_hbm, out_hbm, accum, gsem,
                     srcbuf, dstbuf, gbuf, zbuf):
    c = lax.axis_index("c")
    s = lax.axis_index("s")

    @pl.loop(0, 16 * 8)
    def _(i):
      zbuf[i // 8, pl.ds((i % 8) * 16, 16)] = jnp.zeros((16,), jnp.float32)

    lo = c * NHALF
    lane = lax.iota(jnp.int32, 16)
    trash_src = N + ((s * 16 + lane) % (N_TAB - N))
    trash_dst = NHALF + s * 16 + lane

    for f in range(n_tabs):
      tabf = tab_hbm if n_tabs == 1 else tab_hbm.at[f]
      for k in range(21):
        pltpu.sync_copy(zbuf, accum.at[pl.ds(s * 336 + k * 16, 16)])
      plsc.subcore_barrier()

      for chunk in range(2):
        base = s * rpt + chunk * ch
        pltpu.sync_copy(src_hbm.at[pl.ds(base, ch)], srcbuf)
        pltpu.sync_copy(dst_hbm.at[pl.ds(base, ch)], dstbuf)

        @pl.loop(0, ch * 8)
        def _(i):
          r = i // 8
          g = (i % 8) * 16
          src16 = srcbuf[r, pl.ds(g, 16)]
          dst16 = dstbuf[r, pl.ds(g, 16)]
          local = dst16 - lo
          m = (local >= 0) & (local < NHALF)
          srcbuf[r, pl.ds(g, 16)] = jnp.where(m, src16, trash_src)
          dstbuf[r, pl.ds(g, 16)] = jnp.where(m, local, trash_dst)

        for j in range(4):
          pltpu.async_copy(tabf.at[srcbuf.at[j]], gbuf.at[j], gsem.at[j])

        def process(p, b0):
          for j in range(2):
            b = b0 + j
            blk = p * 2 + j
            pltpu.make_async_copy(tabf.at[srcbuf.at[blk]], gbuf.at[b],
                                  gsem.at[b]).wait()
            pltpu.sync_copy(gbuf.at[b], accum.at[dstbuf.at[blk]], add=True)

            @pl.when(p < (ch // 2) - 2)
            def _():
              pltpu.async_copy(tabf.at[srcbuf.at[blk + 4]], gbuf.at[b],
                               gsem.at[b])

        @pl.loop(0, ch // 2)
        def _(p):
          @pl.when(p % 2 == 0)
          def _():
            process(p, 0)

          @pl.when(p % 2 == 1)
          def _():
            process(p, 2)

      plsc.subcore_barrier()
      if n_tabs == 1:
        pltpu.sync_copy(accum.at[pl.ds(s * 336, 336)],
                        out_hbm.at[c, pl.ds(s * 336, 336)])
      else:
        pltpu.sync_copy(accum.at[pl.ds(s * 336, 336)],
                        out_hbm.at[c, f, pl.ds(s * 336, 336)])

  return body


def _sc_scatter(tab, srcb, dstb, n_tabs):
  out_shape = ((NC, ACC_ROWS, 128) if n_tabs == 1
               else (NC, n_tabs, ACC_ROWS, 128))
  kern = pl.kernel(
      _make_scatter_body(n_tabs),
      out_type=jax.ShapeDtypeStruct(out_shape, jnp.float32),
      mesh=_mesh,
      scratch_types=[
          pltpu.VMEM_SHARED((ACC_ROWS, 128), jnp.float32),
          pltpu.SemaphoreType.DMA((4,)),
      ],
      compiler_params=_sc_params,
  )
  return kern(tab, srcb, dstb)


def _dot(a, b):
  return lax.dot_general(a, b, (((1,), (0,)), ((), ())),
                         precision=lax.Precision.HIGHEST,
                         preferred_element_type=jnp.float32)


def _degt_body(deg_ref, o_ref):
  r = lax.broadcasted_iota(jnp.int32, (64, 4), 0) // 16
  j = lax.broadcasted_iota(jnp.int32, (64, 4), 1)
  sel = (r == j).astype(jnp.float32)
  o_ref[...] = lax.dot_general(deg_ref[...], sel, (((0,), (0,)), ((), ())),
                               precision=lax.Precision.HIGHEST,
                               preferred_element_type=jnp.float32)


def _prep_body(x_ref, degt_ref, o_ref):
  ns1 = lax.rsqrt(jnp.maximum(degt_ref[:, 0:1], 1.0))
  o_ref[...] = x_ref[...] * ns1


def _mid_body(p_ref, degt_ref, w1_ref, b1_ref, o_ref):
  h = _dot(p_ref[0], w1_ref[...])
  nd1 = lax.rsqrt(jnp.maximum(degt_ref[:, 1:2], 1.0))
  ns2 = lax.rsqrt(jnp.maximum(degt_ref[:, 2:3], 1.0))
  xn2 = (h * nd1 + b1_ref[...]) * ns2
  o_ref[0] = xn2[:, :128]
  o_ref[1] = xn2[:, 128:]


def _final_body(a_ref, degt_ref, w2_ref, b2_ref, wfc_ref, bfc_ref, o_ref):
  h2 = _dot(a_ref[0, 0], w2_ref[0:128, :]) + _dot(a_ref[0, 1], w2_ref[128:256, :])
  nd2 = lax.rsqrt(jnp.maximum(degt_ref[:, 3:4], 1.0))
  rst = h2 * nd2 + b2_ref[...]
  o_ref[...] = _dot(rst, wfc_ref[...]) + bfc_ref[...]


def _tc_degt(deg):
  return pl.pallas_call(
      _degt_body,
      out_shape=jax.ShapeDtypeStruct((N_BINS, 4), jnp.float32))(deg)


_BR = 1280


def _tc_prep(x_pad, degt):
  return pl.pallas_call(
      _prep_body,
      grid=(N_TAB // _BR,),
      in_specs=[
          pl.BlockSpec((_BR, 128), lambda i: (i, 0)),
          pl.BlockSpec((_BR, 4), lambda i: (i, 0)),
      ],
      out_specs=pl.BlockSpec((_BR, 128), lambda i: (i, 0)),
      out_shape=jax.ShapeDtypeStruct((N_TAB, 128), jnp.float32))(x_pad, degt)


def _tc_mid(p, degt, W1, b1):
  return pl.pallas_call(
      _mid_body,
      grid=(N_TAB // _BR,),
      in_specs=[
          pl.BlockSpec((1, _BR, 128), lambda i: (i // 4, i % 4, 0)),
          pl.BlockSpec((_BR, 4), lambda i: (i, 0)),
          pl.BlockSpec((128, 256), lambda i: (0, 0)),
          pl.BlockSpec((1, 256), lambda i: (0, 0)),
      ],
      out_specs=pl.BlockSpec((2, _BR, 128), lambda i: (0, i, 0)),
      out_shape=jax.ShapeDtypeStruct((2, N_TAB, 128), jnp.float32))(
          p, degt, W1, b1)


def _tc_final(a, degt, W2, b2, Wfc, bfc):
  return pl.pallas_call(
      _final_body,
      grid=(N_TAB // _BR,),
      in_specs=[
          pl.BlockSpec((1, 2, _BR, 128), lambda i: (i // 4, 0, i % 4, 0)),
          pl.BlockSpec((_BR, 4), lambda i: (i, 0)),
          pl.BlockSpec((256, 256), lambda i: (0, 0)),
          pl.BlockSpec((1, 256), lambda i: (0, 0)),
          pl.BlockSpec((256, 128), lambda i: (0, 0)),
          pl.BlockSpec((1, 128), lambda i: (0, 0)),
      ],
      out_specs=pl.BlockSpec((_BR, 128), lambda i: (i, 0)),
      out_shape=jax.ShapeDtypeStruct((N_TAB, D_OUT), jnp.float32))(
          a, degt, W2, b2, Wfc, bfc)


@jax.jit
def _run(x, edge_index1, edge_index2, W1, b1, W2, b2, Wfc, bfc):
  i32 = jnp.int32
  pad = N + jnp.arange(E_PAD - E, dtype=i32) % (N_TAB - N)

  def prep_idx(v):
    return jnp.concatenate([v.astype(i32), pad]).reshape(EB, BLK)

  s1, d1 = prep_idx(edge_index1[0]), prep_idx(edge_index1[1])
  s2, d2 = prep_idx(edge_index2[0]), prep_idx(edge_index2[1])

  idx_all = jnp.stack([jnp.stack([s1, d1]), jnp.stack([s2, d2])])
  deg = _sc_hist(idx_all)
  degt = _tc_degt(deg.reshape(4 * NS, N_BINS))

  x_pad = jnp.zeros((N_TAB, D_IN), jnp.float32).at[:N].set(x)
  xn = _tc_prep(x_pad, degt)

  p = _sc_scatter(xn, s1, d1, n_tabs=1)
  xh = _tc_mid(p, degt, W1, b1.reshape(1, -1))

  a = _sc_scatter(xh, s2, d2, n_tabs=2)
  out = _tc_final(a, degt, W2, b2.reshape(1, -1), Wfc, bfc.reshape(1, -1))
  return out[:N]


def kernel(x, edge_index1, edge_index2, W1, b1, W2, b2, Wfc, bfc):
  return _run(x, edge_index1, edge_index2, W1, b1, W2, b2, Wfc, bfc)


# speedup vs baseline: 1.3393x; 1.3393x over previous
import dataclasses
import functools

import jax
import jax.numpy as jnp
from jax import lax
from jax.experimental import pallas as pl
from jax.experimental.pallas import tpu as pltpu
from jax.experimental.pallas import tpu_sc as plsc

N = 10000
E = 320000
D_IN = 128
D_HID = 256
D_OUT = 128

NC, NS = 2, 16
BLK = 128
EB = 2560
E_PAD = EB * BLK
N_TAB = 10240
N_BINS = 10240

_mesh = plsc.VectorSubcoreMesh(
    core_axis_name="c", subcore_axis_name="s", num_cores=NC, num_subcores=NS)

_sc_params = pltpu.CompilerParams()
if "needs_layout_passes" in pltpu.CompilerParams.__dataclass_fields__:
  _sc_params = dataclasses.replace(_sc_params, needs_layout_passes=False)


def _hist_body(idx_hbm, out_hbm, idxbuf, histbuf):
  c = lax.axis_index("c")
  s = lax.axis_index("s")
  rpt = EB // NS
  ones16 = jnp.full((16,), 1.0, jnp.float32)

  for k in (0, 1):
    @pl.loop(0, N_BINS // 16)
    def _(i):
      histbuf[pl.ds(i * 16, 16)] = jnp.zeros((16,), jnp.float32)

    pltpu.sync_copy(idx_hbm.at[c, k, pl.ds(s * rpt, rpt)], idxbuf)

    @pl.loop(0, rpt * 8)
    def _(i):
      idx16 = idxbuf[i // 8, pl.ds((i % 8) * 16, 16)]
      plsc.addupdate_scatter(histbuf, [idx16], ones16)

    pltpu.sync_copy(histbuf, out_hbm.at[c, k, s])


def _sc_hist(idx_all):
  kern = pl.kernel(
      _hist_body,
      out_type=jax.ShapeDtypeStruct((2, 2, NS, N_BINS), jnp.float32),
      mesh=_mesh,
      scratch_types=[
          pltpu.VMEM((EB // NS, BLK), jnp.int32),
          pltpu.VMEM((N_BINS,), jnp.float32),
      ],
      compiler_params=_sc_params,
  )
  return kern(idx_all)


LCAP = 164 * 128


def _compact_body(idx_hbm, lsrc_out, ldst_out, cnt_out):
  pl.run_scoped(
      functools.partial(_compact_inner, idx_hbm, lsrc_out, ldst_out, cnt_out),
      pltpu.VMEM((16, BLK), jnp.int32),
      pltpu.VMEM((16, BLK), jnp.int32),
      pltpu.VMEM((LCAP,), jnp.int32),
      pltpu.VMEM((LCAP,), jnp.int32),
      pltpu.VMEM((LCAP,), jnp.int32),
      pltpu.VMEM((LCAP,), jnp.int32),
      pltpu.VMEM((128,), jnp.int32),
  )


def _compact_inner(idx_hbm, lsrc_out, ldst_out, cnt_out,
                   sidx, didx, ls0, ld0, ls1, ld1, cbuf):
  c = lax.axis_index("c")
  s = lax.axis_index("s")
  rpt = EB // NS
  lane = lax.iota(jnp.int32, 16)
  trash_src = N + ((s * 16 + lane) % (N_TAB - N))
  trash_dst = NHALF + s * 16 + lane

  def chunk(k, carry):
    p0, p1 = carry
    pltpu.sync_copy(idx_hbm.at[c, 0, pl.ds(s * rpt + k * 16, 16)], sidx)
    pltpu.sync_copy(idx_hbm.at[c, 1, pl.ds(s * rpt + k * 16, 16)], didx)

    def grp(i, pp):
      q0, q1 = pp
      r = i // 8
      g = (i % 8) * 16
      src16 = sidx[r, pl.ds(g, 16)]
      dst16 = didx[r, pl.ds(g, 16)]
      m0 = dst16 < NHALF
      i0 = m0.astype(jnp.int32)
      cs0 = plsc.cumsum(i0)
      n0 = jnp.max(cs0)
      e0 = cs0 - i0
      plsc.store_scatter(ls0, [q0 + e0], src16, mask=m0)
      plsc.store_scatter(ld0, [q0 + e0], dst16, mask=m0)
      m1 = jnp.logical_not(m0)
      i1 = jnp.int32(1) - i0
      e1 = plsc.cumsum(i1) - i1
      plsc.store_scatter(ls1, [q1 + e1], src16, mask=m1)
      plsc.store_scatter(ld1, [q1 + e1], dst16 - NHALF, mask=m1)
      return (q0 + n0, q1 + (16 - n0))

    return lax.fori_loop(0, 16 * 8, grp, (p0, p1))

  p0, p1 = lax.fori_loop(0, 10, chunk, (jnp.int32(0), jnp.int32(0)))

  @pl.loop(0, 16)
  def _(k):
    off = k * 16 + lane
    plsc.store_scatter(ls0, [p0 + off], trash_src)
    plsc.store_scatter(ld0, [p0 + off], trash_dst)
    plsc.store_scatter(ls1, [p1 + off], trash_src)
    plsc.store_scatter(ld1, [p1 + off], trash_dst)

  nb0 = 2 * ((p0 + 255) // 256)
  nb1 = 2 * ((p1 + 255) // 256)

  @pl.loop(0, 8)
  def _(k):
    cbuf[pl.ds(k * 16, 16)] = jnp.where(lane == 0, nb0,
                                        jnp.where(lane == 1, nb1, 0))

  pltpu.sync_copy(ls0, lsrc_out.at[c, 0, pl.ds(s * LCAP, LCAP)])
  pltpu.sync_copy(ld0, ldst_out.at[c, 0, pl.ds(s * LCAP, LCAP)])
  pltpu.sync_copy(ls1, lsrc_out.at[c, 1, pl.ds(s * LCAP, LCAP)])
  pltpu.sync_copy(ld1, ldst_out.at[c, 1, pl.ds(s * LCAP, LCAP)])
  pltpu.sync_copy(cbuf, cnt_out.at[c, pl.ds(s * 128, 128)])


def _sc_compact(idx_all):
  kern = pl.kernel(
      _compact_body,
      out_type=(
          jax.ShapeDtypeStruct((2, 2, NS * LCAP), jnp.int32),
          jax.ShapeDtypeStruct((2, 2, NS * LCAP), jnp.int32),
          jax.ShapeDtypeStruct((2, NS * 128), jnp.int32),
      ),
      mesh=_mesh,
      scratch_types=[],
      compiler_params=_sc_params,
  )
  return kern(idx_all)


NHALF = N_TAB // 2
ACC_ROWS = 5376


def _make_scatter_body(n_tabs):

  def body(tab_hbm, lsrc_hbm, ldst_hbm, cnt_hbm, out_hbm, accum, gsem):
    pl.run_scoped(
        functools.partial(_scatter_inner, tab_hbm, lsrc_hbm, ldst_hbm,
                          cnt_hbm, out_hbm, accum, gsem),
        pltpu.VMEM((LCAP // 128, BLK), jnp.int32),
        pltpu.VMEM((LCAP // 128, BLK), jnp.int32),
        pltpu.VMEM((2, BLK, 128), jnp.float32),
        pltpu.VMEM((16, 128), jnp.float32),
        pltpu.VMEM((128,), jnp.int32),
    )

  def _scatter_inner(tab_hbm, lsrc_hbm, ldst_hbm, cnt_hbm, out_hbm, accum,
                     gsem, srcbuf, dstbuf, gbuf, zbuf, cbuf):
    c = lax.axis_index("c")
    s = lax.axis_index("s")

    @pl.loop(0, 16 * 8)
    def _(i):
      zbuf[i // 8, pl.ds((i % 8) * 16, 16)] = jnp.zeros((16,), jnp.float32)

    pltpu.sync_copy(lsrc_hbm.at[c, s], srcbuf)
    pltpu.sync_copy(ldst_hbm.at[c, s], dstbuf)
    pltpu.sync_copy(cnt_hbm.at[pl.ds(s * 128, 128)], cbuf)
    lane = lax.iota(jnp.int32, 16)
    cv = cbuf[pl.ds(0, 16)]
    nb = jnp.minimum(jnp.max(jnp.where(lane == c, cv, 0)), LCAP // 128)

    for f in range(n_tabs):
      tabf = tab_hbm if n_tabs == 1 else tab_hbm.at[f]
      for k in range(21):
        pltpu.sync_copy(zbuf, accum.at[pl.ds(s * 336 + k * 16, 16)])
      plsc.subcore_barrier()

      @pl.loop(0, nb // 2)
      def _(i2):
        gd = [pltpu.async_copy(tabf.at[srcbuf.at[i2 * 2 + j]], gbuf.at[j],
                               gsem.at[j]) for j in range(2)]
        for j in range(2):
          gd[j].wait()
          pltpu.sync_copy(gbuf.at[j], accum.at[dstbuf.at[i2 * 2 + j]],
                          add=True)

      plsc.subcore_barrier()
      if n_tabs == 1:
        pltpu.sync_copy(accum.at[pl.ds(s * 336, 336)],
                        out_hbm.at[c, pl.ds(s * 336, 336)])
      else:
        pltpu.sync_copy(accum.at[pl.ds(s * 336, 336)],
                        out_hbm.at[c, f, pl.ds(s * 336, 336)])

  return body


def _sc_scatter(tab, lsrc, ldst, cnt, n_tabs):
  out_shape = ((NC, ACC_ROWS, 128) if n_tabs == 1
               else (NC, n_tabs, ACC_ROWS, 128))
  kern = pl.kernel(
      _make_scatter_body(n_tabs),
      out_type=jax.ShapeDtypeStruct(out_shape, jnp.float32),
      mesh=_mesh,
      scratch_types=[
          pltpu.VMEM_SHARED((ACC_ROWS, 128), jnp.float32),
          pltpu.SemaphoreType.DMA((2,)),
      ],
      compiler_params=_sc_params,
  )
  return kern(tab, lsrc, ldst, cnt)


def _dot(a, b):
  return lax.dot_general(a, b, (((1,), (0,)), ((), ())),
                         precision=lax.Precision.HIGHEST,
                         preferred_element_type=jnp.float32)


def _degt_body(deg_ref, o_ref):
  r = lax.broadcasted_iota(jnp.int32, (64, 4), 0) // 16
  j = lax.broadcasted_iota(jnp.int32, (64, 4), 1)
  sel = (r == j).astype(jnp.float32)
  o_ref[...] = lax.dot_general(deg_ref[...], sel, (((0,), (0,)), ((), ())),
                               precision=lax.Precision.HIGHEST,
                               preferred_element_type=jnp.float32)


def _prep_body(x_ref, degt_ref, o_ref):
  ns1 = lax.rsqrt(jnp.maximum(degt_ref[:, 0:1], 1.0))
  o_ref[...] = x_ref[...] * ns1


def _mid_body(p_ref, degt_ref, w1_ref, b1_ref, o_ref):
  h = _dot(p_ref[0], w1_ref[...])
  nd1 = lax.rsqrt(jnp.maximum(degt_ref[:, 1:2], 1.0))
  ns2 = lax.rsqrt(jnp.maximum(degt_ref[:, 2:3], 1.0))
  xn2 = (h * nd1 + b1_ref[...]) * ns2
  o_ref[0] = xn2[:, :128]
  o_ref[1] = xn2[:, 128:]


def _final_body(a_ref, degt_ref, w2_ref, b2_ref, wfc_ref, bfc_ref, o_ref):
  h2 = _dot(a_ref[0, 0], w2_ref[0:128, :]) + _dot(a_ref[0, 1], w2_ref[128:256, :])
  nd2 = lax.rsqrt(jnp.maximum(degt_ref[:, 3:4], 1.0))
  rst = h2 * nd2 + b2_ref[...]
  o_ref[...] = _dot(rst, wfc_ref[...]) + bfc_ref[...]


def _tc_degt(deg):
  return pl.pallas_call(
      _degt_body,
      out_shape=jax.ShapeDtypeStruct((N_BINS, 4), jnp.float32))(deg)


_BR = 1280


def _tc_prep(x_pad, degt):
  return pl.pallas_call(
      _prep_body,
      grid=(N_TAB // _BR,),
      in_specs=[
          pl.BlockSpec((_BR, 128), lambda i: (i, 0)),
          pl.BlockSpec((_BR, 4), lambda i: (i, 0)),
      ],
      out_specs=pl.BlockSpec((_BR, 128), lambda i: (i, 0)),
      out_shape=jax.ShapeDtypeStruct((N_TAB, 128), jnp.float32))(x_pad, degt)


def _tc_mid(p, degt, W1, b1):
  return pl.pallas_call(
      _mid_body,
      grid=(N_TAB // _BR,),
      in_specs=[
          pl.BlockSpec((1, _BR, 128), lambda i: (i // 4, i % 4, 0)),
          pl.BlockSpec((_BR, 4), lambda i: (i, 0)),
          pl.BlockSpec((128, 256), lambda i: (0, 0)),
          pl.BlockSpec((1, 256), lambda i: (0, 0)),
      ],
      out_specs=pl.BlockSpec((2, _BR, 128), lambda i: (0, i, 0)),
      out_shape=jax.ShapeDtypeStruct((2, N_TAB, 128), jnp.float32))(
          p, degt, W1, b1)


def _tc_final(a, degt, W2, b2, Wfc, bfc):
  return pl.pallas_call(
      _final_body,
      grid=(N_TAB // _BR,),
      in_specs=[
          pl.BlockSpec((1, 2, _BR, 128), lambda i: (i // 4, 0, i % 4, 0)),
          pl.BlockSpec((_BR, 4), lambda i: (i, 0)),
          pl.BlockSpec((256, 256), lambda i: (0, 0)),
          pl.BlockSpec((1, 256), lambda i: (0, 0)),
          pl.BlockSpec((256, 128), lambda i: (0, 0)),
          pl.BlockSpec((1, 128), lambda i: (0, 0)),
      ],
      out_specs=pl.BlockSpec((_BR, 128), lambda i: (i, 0)),
      out_shape=jax.ShapeDtypeStruct((N_TAB, D_OUT), jnp.float32))(
          a, degt, W2, b2, Wfc, bfc)


@jax.jit
def _run(x, edge_index1, edge_index2, W1, b1, W2, b2, Wfc, bfc):
  i32 = jnp.int32
  pad = N + jnp.arange(E_PAD - E, dtype=i32) % (N_TAB - N)

  def prep_idx(v):
    return jnp.concatenate([v.astype(i32), pad]).reshape(EB, BLK)

  s1, d1 = prep_idx(edge_index1[0]), prep_idx(edge_index1[1])
  s2, d2 = prep_idx(edge_index2[0]), prep_idx(edge_index2[1])

  idx_all = jnp.stack([jnp.stack([s1, d1]), jnp.stack([s2, d2])])
  deg = _sc_hist(idx_all)
  lsrc, ldst, cnt = _sc_compact(idx_all)
  lsrc = lsrc.reshape(2, 2, NS, LCAP // 128, 128)
  ldst = ldst.reshape(2, 2, NS, LCAP // 128, 128)
  degt = _tc_degt(deg.reshape(4 * NS, N_BINS))

  x_pad = jnp.zeros((N_TAB, D_IN), jnp.float32).at[:N].set(x)
  xn = _tc_prep(x_pad, degt)

  p = _sc_scatter(xn, lsrc[0], ldst[0], cnt[0], n_tabs=1)
  xh = _tc_mid(p, degt, W1, b1.reshape(1, -1))

  a = _sc_scatter(xh, lsrc[1], ldst[1], cnt[1], n_tabs=2)
  out = _tc_final(a, degt, W2, b2.reshape(1, -1), Wfc, bfc.reshape(1, -1))
  return out[:N]


def kernel(x, edge_index1, edge_index2, W1, b1, W2, b2, Wfc, bfc):
  return _run(x, edge_index1, edge_index2, W1, b1, W2, b2, Wfc, bfc)
